# baseline (speedup 1.0000x reference)
import functools

import jax
import jax.numpy as jnp
from jax import lax
from jax.experimental import pallas as pl
from jax.experimental.pallas import tpu as pltpu
from jax.experimental.pallas import tpu_sc as plsc

_H = 8
_B = 8
_DH = 16
_NT = 32
_EB = 128
_E_PAD = 81920
_NB = _E_PAD // (_NT * _EB)


def _mm_bias(x, w, b):
    n, k = x.shape
    p = w.shape[1]
    bm = 1000
    assert n % bm == 0

    def body(x_ref, w_ref, b_ref, o_ref):
        o_ref[...] = (
            jnp.dot(x_ref[...], w_ref[...], preferred_element_type=jnp.float32)
            + b_ref[...]
        )

    return pl.pallas_call(
        body,
        grid=(n // bm,),
        in_specs=[
            pl.BlockSpec((bm, k), lambda i: (i, 0)),
            pl.BlockSpec((k, p), lambda i: (0, 0)),
            pl.BlockSpec((1, p), lambda i: (0, 0)),
        ],
        out_specs=pl.BlockSpec((bm, p), lambda i: (i, 0)),
        out_shape=jax.ShapeDtypeStruct((n, p), jnp.float32),
    )(x, w, b.reshape(1, p))


def _seg_sum_cnt(basesT, srcp, dstp, nd_pad):
    rpt = nd_pad // 16
    zeros_acc = jnp.zeros((nd_pad, 16), jnp.float32)
    ones_rows = jnp.ones((_EB, 16), jnp.float32)
    mesh = plsc.VectorSubcoreMesh(core_axis_name="c", subcore_axis_name="s")

    @functools.partial(
        pl.kernel,
        mesh=mesh,
        compiler_params=pltpu.CompilerParams(use_tc_tiling_on_sc=False),
        out_type=[
            jax.ShapeDtypeStruct((2, 8, nd_pad, 16), jnp.float32),
            jax.ShapeDtypeStruct((2, nd_pad, 16), jnp.float32),
        ],
        scratch_types=[
            pltpu.VMEM_SHARED((nd_pad, 16), jnp.float32),
            pltpu.VMEM((_NB, _EB), jnp.int32),
            pltpu.VMEM((_NB, _EB), jnp.int32),
            pltpu.VMEM((_EB, 16), jnp.float32),
            pltpu.VMEM((_EB, 16), jnp.float32),
            pltpu.SemaphoreType.DMA,
        ],
    )
    def k(basesT_h, srcp_h, dstp_h, zeros_h, ones_h, out_sum, out_cnt,
          acc, srcv, dstv, rowsv, onesv, sem):
        sid = lax.axis_index("s")
        cid = lax.axis_index("c")
        wid = cid * 16 + sid
        r0 = sid * rpt
        pltpu.sync_copy(srcp_h.at[wid], srcv)
        pltpu.sync_copy(dstp_h.at[wid], dstv)
        pltpu.sync_copy(ones_h, onesv)
        pltpu.sync_copy(zeros_h.at[pl.ds(r0, rpt)], acc.at[pl.ds(r0, rpt)])
        plsc.subcore_barrier()

        def cnt_step(j, carry):
            pltpu.sync_copy(onesv, acc.at[dstv.at[j]], add=True)
            return carry

        lax.fori_loop(0, _NB, cnt_step, 0)
        plsc.subcore_barrier()
        pltpu.sync_copy(acc.at[pl.ds(r0, rpt)],
                        out_cnt.at[cid].at[pl.ds(r0, rpt)])
        for f in range(8):
            plsc.subcore_barrier()
            pltpu.sync_copy(zeros_h.at[pl.ds(r0, rpt)], acc.at[pl.ds(r0, rpt)])
            plsc.subcore_barrier()

            def gather_step(j, carry):
                pltpu.async_copy(
                    basesT_h.at[f].at[srcv.at[j]],
                    rowsv, sem).wait()
                pltpu.sync_copy(rowsv, acc.at[dstv.at[j]], add=True)
                return carry

            lax.fori_loop(0, _NB, gather_step, 0)
            plsc.subcore_barrier()
            pltpu.sync_copy(acc.at[pl.ds(r0, rpt)],
                            out_sum.at[cid].at[f].at[pl.ds(r0, rpt)])

    out_sum2, out_cnt2 = k(basesT, srcp, dstp, zeros_acc, ones_rows)
    return out_sum2[0] + out_sum2[1], out_cnt2[0] + out_cnt2[1]


def _pad_edges(src, dst, nd_pad):
    pad = _E_PAD - src.shape[0]
    src = jnp.concatenate(
        [src.astype(jnp.int32), jnp.zeros((pad,), jnp.int32)])
    dst = jnp.concatenate(
        [dst.astype(jnp.int32),
         jnp.full((pad,), nd_pad - 1, jnp.int32)])
    return (src.reshape(_NT, _NB, _EB), dst.reshape(_NT, _NB, _EB))


def kernel(x_author, x_field_of_study, x_institution, x_paper, src_author_affiliated_with_institution, dst_author_affiliated_with_institution, src_institution_to_author, dst_institution_to_author, src_author_writes_paper, dst_author_writes_paper, src_paper_to_author, dst_paper_to_author, src_paper_cites_paper, dst_paper_cites_paper, src_paper_has_topic_field_of_study, dst_paper_has_topic_field_of_study, src_field_of_study_to_paper, dst_field_of_study_to_paper, bases_weight, relW_author_affiliated_with_institution, relb_author_affiliated_with_institution, relW_institution_to_author, relb_institution_to_author, relW_author_writes_paper, relb_author_writes_paper, relW_paper_to_author, relb_paper_to_author, relW_paper_cites_paper, relb_paper_cites_paper, relW_paper_has_topic_field_of_study, relb_paper_has_topic_field_of_study, relW_field_of_study_to_paper, relb_field_of_study_to_paper, rootW_author, rootb_author, rootW_field_of_study, rootb_field_of_study, rootW_institution, rootb_institution, rootW_paper, rootb_paper):
    inp = dict(locals())
    node_types = ['author', 'field_of_study', 'institution', 'paper']
    edge_types = [
        ('author', 'affiliated_with', 'institution'),
        ('institution', 'to', 'author'),
        ('author', 'writes', 'paper'),
        ('paper', 'to', 'author'),
        ('paper', 'cites', 'paper'),
        ('paper', 'has_topic', 'field_of_study'),
        ('field_of_study', 'to', 'paper'),
    ]
    nd_pads = {'author': 50176, 'field_of_study': 10240,
               'institution': 10240, 'paper': 100096}

    zero128 = jnp.zeros((bases_weight.shape[1],), dtype=jnp.float32)
    bases = {nt: _mm_bias(inp['x_' + nt], bases_weight, zero128)
             for nt in node_types}
    basesT = {nt: bases[nt].reshape(-1, 8, 16).transpose(1, 0, 2)
              for nt in node_types}
    root = {}
    for nt in node_types:
        w = _mm_bias(inp['x_' + nt], inp['rootW_' + nt], inp['rootb_' + nt])
        w = w.reshape(-1, _H, _B)
        root[nt] = jnp.matmul(w, bases[nt].reshape(-1, _B, _DH))

    for (s, r, d) in edge_types:
        tag = s + '_' + r + '_' + d
        src = inp['src_' + tag]
        dst = inp['dst_' + tag]
        nd = inp['x_' + d].shape[0]
        nd_pad = nd_pads[d]
        srcp, dstp = _pad_edges(src, dst, nd_pad)
        seg_sum8, cnt16 = _seg_sum_cnt(basesT[s], srcp, dstp, nd_pad)
        seg_sum = seg_sum8.transpose(1, 0, 2).reshape(nd_pad, -1)[:nd]
        cnt = cnt16[:nd, 0]
        agg_mean = seg_sum / jnp.maximum(cnt, 1.0)[:, None]
        msgs = bases[s][src]
        agg_max = jax.ops.segment_max(msgs, dst, num_segments=nd)
        agg_max = jnp.where(cnt[:, None] > 0, agg_max, 0.0)
        agg = jnp.stack([agg_mean, agg_max], axis=1).reshape(-1, 2 * _B, _DH)
        w = _mm_bias(inp['x_' + d], inp['relW_' + tag], inp['relb_' + tag])
        w = w.reshape(-1, _H, 2 * _B)
        root[d] = root[d] + jnp.matmul(w, agg)

    return tuple(root[nt].reshape(-1, _H * _DH) for nt in node_types)
